# Initial kernel scaffold; baseline (speedup 1.0000x reference)
import jax
import jax.numpy as jnp
from jax.experimental import pallas as pl


def kernel(x, W_enc, b_enc, W_dec, b_dec):
    raise NotImplementedError("write your pallas kernel here")


# speedup vs baseline: 6.3303x; 6.3303x over previous
import functools

import jax
import jax.numpy as jnp
from jax.experimental import pallas as pl

N, C, D, K = 4096, 2048, 16384, 64


def _enc_kernel(x_ref, w_ref, benc_ref, bdec_ref, out_ref):
    xc = x_ref[...] - bdec_ref[...]
    acc = jax.lax.dot_general(
        xc, w_ref[...], (((1,), (1,)), ((), ())),
        preferred_element_type=jnp.float32,
        precision=jax.lax.Precision.DEFAULT)
    out_ref[...] = jnp.maximum(acc + benc_ref[...], 0.0)


def _encode(x, W_enc, b_enc, b_dec):
    BN, BD = min(1024, N), 512
    grid = (N // BN, D // BD)
    return pl.pallas_call(
        _enc_kernel,
        grid=grid,
        in_specs=[
            pl.BlockSpec((BN, C), lambda i, j: (i, 0)),
            pl.BlockSpec((BD, C), lambda i, j: (j, 0)),
            pl.BlockSpec((1, BD), lambda i, j: (0, j)),
            pl.BlockSpec((1, C), lambda i, j: (0, 0)),
        ],
        out_specs=pl.BlockSpec((BN, BD), lambda i, j: (i, j)),
        out_shape=jax.ShapeDtypeStruct((N, D), jnp.float32),
    )(x, W_enc, b_enc.reshape(1, D), b_dec.reshape(1, C))


def _topk_kernel(post_ref, out_ref):
    post = post_ref[...]
    vi = jax.lax.bitcast_convert_type(post, jnp.int32)

    def body(b, cand):
        test = cand | (1 << (30 - b))
        cnt = jnp.sum((vi >= test).astype(jnp.int32), axis=1, keepdims=True)
        return jnp.where(cnt >= K, test, cand)

    cand = jax.lax.fori_loop(0, 31, body, jnp.zeros((post.shape[0], 1), jnp.int32))
    out_ref[...] = jnp.where(vi >= cand, post, 0.0)


def _topk_mask(post):
    BN = min(128, N)
    return pl.pallas_call(
        _topk_kernel,
        grid=(N // BN,),
        in_specs=[pl.BlockSpec((BN, D), lambda i: (i, 0))],
        out_specs=pl.BlockSpec((BN, D), lambda i: (i, 0)),
        out_shape=jax.ShapeDtypeStruct((N, D), jnp.float32),
    )(post)


def _dec_kernel(enc_ref, w_ref, bdec_ref, out_ref):
    j = pl.program_id(1)
    acc = jax.lax.dot_general(
        enc_ref[...], w_ref[...], (((1,), (1,)), ((), ())),
        preferred_element_type=jnp.float32,
        precision=jax.lax.Precision.HIGHEST)

    @pl.when(j == 0)
    def _():
        out_ref[...] = acc + bdec_ref[...]

    @pl.when(j != 0)
    def _():
        out_ref[...] += acc


def _decode(encoded, W_dec, b_dec):
    BN, BD = min(1024, N), 512
    grid = (N // BN, D // BD)
    return pl.pallas_call(
        _dec_kernel,
        grid=grid,
        in_specs=[
            pl.BlockSpec((BN, BD), lambda i, j: (i, j)),
            pl.BlockSpec((C, BD), lambda i, j: (0, j)),
            pl.BlockSpec((1, C), lambda i, j: (0, 0)),
        ],
        out_specs=pl.BlockSpec((BN, C), lambda i, j: (i, 0)),
        out_shape=jax.ShapeDtypeStruct((N, C), jnp.float32),
    )(encoded, W_dec, b_dec.reshape(1, C))


@jax.jit
def kernel(x, W_enc, b_enc, W_dec, b_dec):
    post = _encode(x, W_enc, b_enc, b_dec)
    encoded = _topk_mask(post)
    recon = _decode(encoded, W_dec, b_dec)
    return (recon, encoded)
